# baseline (speedup 1.0000x reference)
import functools

import jax
import jax.numpy as jnp
from jax import lax
from jax.experimental import pallas as pl
from jax.experimental.pallas import tpu as pltpu
from jax.experimental.pallas import tpu_sc as plsc

N_PMTS = 494
BATCH = 16384
M = 256
SWITCHING_SIGNAL = 50.0
P_DPE = 0.2
NAN_SAFE_VALUE = 1.0e6

NP_PAD = 512
NW = 32
K_MAX = (N_PMTS + NW - 1) // NW
L = 16
STEPS = BATCH // L
BLK_E = 256
FB_FLAG = 1 << 16


def _tc_encode(pred, observed, coef_row):
    def body(p_ref, x_ref, c_ref, o_ref):
        x = x_ref[...]
        mu = jnp.maximum(p_ref[...], 1e-6)
        c = c_ref[...]

        tx = x / SWITCHING_SIGNAL * float(M)
        tx = jnp.minimum(jnp.maximum(tx, 0.0), float(M - 1))
        xi = tx.astype(jnp.int32)
        tm = mu / SWITCHING_SIGNAL * float(M)
        tm = jnp.minimum(jnp.maximum(tm, 0.0), float(M - 1))
        mi = tm.astype(jnp.int32)
        idx = xi * M + mi

        var = jnp.maximum(mu * c, 1e-12)
        d = x - mu * (1.0 + P_DPE)
        g = (d * d) / var
        nan = (x != x) | (mu != mu)
        fb = jnp.where(nan, NAN_SAFE_VALUE, g)
        fbb = lax.bitcast_convert_type(fb.astype(jnp.bfloat16), jnp.uint16)
        fbb = fbb.astype(jnp.int32) | FB_FLAG

        use_lut = (x < SWITCHING_SIGNAL) & (mu < SWITCHING_SIGNAL)
        enc = jnp.where(use_lut, idx, fbb)
        enc = jnp.concatenate(
            [enc, jnp.zeros((BLK_E, NP_PAD - N_PMTS), jnp.int32)], axis=1)
        o_ref[...] = enc.T

    return pl.pallas_call(
        body,
        grid=(BATCH // BLK_E,),
        in_specs=[
            pl.BlockSpec((BLK_E, N_PMTS), lambda i: (i, 0)),
            pl.BlockSpec((BLK_E, N_PMTS), lambda i: (i, 0)),
            pl.BlockSpec((1, N_PMTS), lambda i: (0, 0)),
        ],
        out_specs=pl.BlockSpec((NP_PAD, BLK_E), lambda i: (0, i)),
        out_shape=jax.ShapeDtypeStruct((NP_PAD, BATCH), jnp.int32),
    )(pred, observed, coef_row)


def _sc_stage(enc, lut):
    mesh = plsc.VectorSubcoreMesh(core_axis_name="c", subcore_axis_name="s")

    @functools.partial(
        pl.kernel,
        out_type=jax.ShapeDtypeStruct((NW, BATCH), jnp.float32),
        mesh=mesh,
        scratch_types=[
            pltpu.VMEM((M * M,), jnp.float32),
            pltpu.VMEM((BATCH,), jnp.int32),
            pltpu.VMEM((BATCH,), jnp.float32),
        ],
        compiler_params=pltpu.CompilerParams(needs_layout_passes=False),
    )
    def body(enc_hbm, lut_hbm, out_hbm, tab_v, enc_v, acc_v):
        wid = lax.axis_index("s") * 2 + lax.axis_index("c")

        for k in range(K_MAX):
            p = wid + k * NW

            @pl.when(p < N_PMTS)
            def _():
                pltpu.sync_copy(lut_hbm.at[p], tab_v)
                pltpu.sync_copy(enc_hbm.at[p], enc_v)

                @pl.loop(0, STEPS, unroll=8)
                def _step(i):
                    base = i * L
                    v = enc_v[pl.ds(base, L)]
                    is_lut = v < FB_FLAG
                    idxv = jnp.minimum(v, FB_FLAG - 1)
                    val = plsc.load_gather(tab_v, [idxv])
                    fbv = plsc.bitcast(v << 16, jnp.float32)
                    r = jnp.where(is_lut, val, fbv)
                    if k == 0:
                        acc_v[pl.ds(base, L)] = r
                    else:
                        acc_v[pl.ds(base, L)] = acc_v[pl.ds(base, L)] + r

        pltpu.sync_copy(acc_v, out_hbm.at[wid])

    return body(enc, lut)


def _tc_reduce(partials):
    def body(p_ref, o_ref):
        o_ref[...] = jnp.sum(p_ref[...], axis=0, keepdims=True)

    return pl.pallas_call(
        body,
        out_shape=jax.ShapeDtypeStruct((1, BATCH), jnp.float32),
    )(partials)


def kernel(pred, observed, std, L_table):
    lut = L_table.reshape(N_PMTS, M * M)
    coef = (1.0 + P_DPE) ** 2 + P_DPE * (1.0 - P_DPE) + std * std
    enc = _tc_encode(pred, observed, coef.reshape(1, N_PMTS))
    partials = _sc_stage(enc, lut)
    return _tc_reduce(partials).reshape(BATCH)


# speedup vs baseline: 1.4486x; 1.4486x over previous
import functools

import jax
import jax.numpy as jnp
from jax import lax
from jax.experimental import pallas as pl
from jax.experimental.pallas import tpu as pltpu
from jax.experimental.pallas import tpu_sc as plsc

N_PMTS = 494
BATCH = 16384
M = 256
SWITCHING_SIGNAL = 50.0
P_DPE = 0.2
NAN_SAFE_VALUE = 1.0e6

NP_PAD = 512
NW = 32
K_MAX = (N_PMTS + NW - 1) // NW
L = 16
STEPS = BATCH // L
BLK_E = 1024
FB_FLAG = 1 << 16


def _tc_encode(pred, observed, coef_row):
    def body(p_ref, x_ref, c_ref, o_ref):
        x = x_ref[...]
        mu = jnp.maximum(p_ref[...], 1e-6)
        c = c_ref[...]

        tx = x / SWITCHING_SIGNAL * float(M)
        tx = jnp.minimum(jnp.maximum(tx, 0.0), float(M - 1))
        xi = tx.astype(jnp.int32)
        tm = mu / SWITCHING_SIGNAL * float(M)
        tm = jnp.minimum(jnp.maximum(tm, 0.0), float(M - 1))
        mi = tm.astype(jnp.int32)
        idx = xi * M + mi

        var = jnp.maximum(mu * c, 1e-12)
        d = x - mu * (1.0 + P_DPE)
        g = (d * d) / var
        nan = (x != x) | (mu != mu)
        fb = jnp.where(nan, NAN_SAFE_VALUE, g)
        fbb = lax.bitcast_convert_type(fb.astype(jnp.bfloat16), jnp.uint16)
        fbb = fbb.astype(jnp.int32) | FB_FLAG

        use_lut = (x < SWITCHING_SIGNAL) & (mu < SWITCHING_SIGNAL)
        enc = jnp.where(use_lut, idx, fbb)
        enc = jnp.concatenate(
            [enc, jnp.zeros((BLK_E, NP_PAD - N_PMTS), jnp.int32)], axis=1)
        o_ref[...] = enc.T.reshape(NP_PAD, BLK_E // 128, 128)

    return pl.pallas_call(
        body,
        grid=(BATCH // BLK_E,),
        in_specs=[
            pl.BlockSpec((BLK_E, N_PMTS), lambda i: (i, 0)),
            pl.BlockSpec((BLK_E, N_PMTS), lambda i: (i, 0)),
            pl.BlockSpec((1, N_PMTS), lambda i: (0, 0)),
        ],
        out_specs=pl.BlockSpec((NP_PAD, BLK_E // 128, 128), lambda i: (0, i, 0)),
        out_shape=jax.ShapeDtypeStruct((NP_PAD, BATCH // 128, 128), jnp.int32),
    )(pred, observed, coef_row)


def _sc_stage(enc, lut):
    mesh = plsc.VectorSubcoreMesh(core_axis_name="c", subcore_axis_name="s")

    @functools.partial(
        pl.kernel,
        out_type=jax.ShapeDtypeStruct((NW, BATCH), jnp.float32),
        mesh=mesh,
        scratch_types=[
            pltpu.VMEM((M * M,), jnp.float32),
            pltpu.VMEM((BATCH // 128, 128), jnp.int32),
            pltpu.VMEM((BATCH,), jnp.float32),
        ],
        compiler_params=pltpu.CompilerParams(needs_layout_passes=False),
    )
    def body(enc_hbm, lut_hbm, out_hbm, tab_v, enc_v, acc_v):
        wid = lax.axis_index("s") * 2 + lax.axis_index("c")

        for k in range(K_MAX):
            p = wid + k * NW

            @pl.when(p < N_PMTS)
            def _():
                pltpu.sync_copy(lut_hbm.at[p], tab_v)
                pltpu.sync_copy(enc_hbm.at[p], enc_v)

                @plsc.parallel_loop(0, BATCH // 128, unroll=2)
                def _row(rr):
                    for cc in range(8):
                        base = rr * 128 + cc * L
                        v = enc_v[rr, pl.ds(cc * L, L)]
                        is_lut = v < FB_FLAG
                        idxv = jnp.minimum(v, FB_FLAG - 1)
                        val = plsc.load_gather(tab_v, [idxv])
                        fbv = plsc.bitcast(v << 16, jnp.float32)
                        r = jnp.where(is_lut, val, fbv)
                        if k == 0:
                            acc_v[pl.ds(base, L)] = r
                        else:
                            acc_v[pl.ds(base, L)] = acc_v[pl.ds(base, L)] + r

        pltpu.sync_copy(acc_v, out_hbm.at[wid])

    return body(enc, lut)


def _tc_reduce(partials):
    def body(p_ref, o_ref):
        o_ref[...] = jnp.sum(p_ref[...], axis=0, keepdims=True)

    return pl.pallas_call(
        body,
        out_shape=jax.ShapeDtypeStruct((1, BATCH), jnp.float32),
    )(partials)


def kernel(pred, observed, std, L_table):
    lut = L_table.reshape(N_PMTS, M * M)
    coef = (1.0 + P_DPE) ** 2 + P_DPE * (1.0 - P_DPE) + std * std
    enc = _tc_encode(pred, observed, coef.reshape(1, N_PMTS))
    partials = _sc_stage(enc, lut)
    return _tc_reduce(partials).reshape(BATCH)


# speedup vs baseline: 1.5418x; 1.0643x over previous
import functools

import jax
import jax.numpy as jnp
from jax import lax
from jax.experimental import pallas as pl
from jax.experimental.pallas import tpu as pltpu
from jax.experimental.pallas import tpu_sc as plsc

N_PMTS = 494
BATCH = 16384
M = 256
SWITCHING_SIGNAL = 50.0
P_DPE = 0.2
NAN_SAFE_VALUE = 1.0e6

NP_PAD = 512
NW = 32
K_MAX = (N_PMTS + NW - 1) // NW
L = 16
STEPS = BATCH // L
BLK_E = 1024
FB_FLAG = 1 << 16


def _tc_encode(pred, observed, coef_row):
    def body(p_ref, x_ref, c_ref, o_ref):
        x = x_ref[...]
        mu = jnp.maximum(p_ref[...], 1e-6)
        c = c_ref[...]

        tx = x / SWITCHING_SIGNAL * float(M)
        tx = jnp.minimum(jnp.maximum(tx, 0.0), float(M - 1))
        xi = tx.astype(jnp.int32)
        tm = mu / SWITCHING_SIGNAL * float(M)
        tm = jnp.minimum(jnp.maximum(tm, 0.0), float(M - 1))
        mi = tm.astype(jnp.int32)
        idx = xi * M + mi

        var = jnp.maximum(mu * c, 1e-12)
        d = x - mu * (1.0 + P_DPE)
        g = (d * d) / var
        nan = (x != x) | (mu != mu)
        fb = jnp.where(nan, NAN_SAFE_VALUE, g)
        fbb = lax.bitcast_convert_type(fb.astype(jnp.bfloat16), jnp.uint16)
        fbb = fbb.astype(jnp.int32) | FB_FLAG

        use_lut = (x < SWITCHING_SIGNAL) & (mu < SWITCHING_SIGNAL)
        enc = jnp.where(use_lut, idx, fbb)
        enc = jnp.concatenate(
            [enc, jnp.zeros((BLK_E, NP_PAD - N_PMTS), jnp.int32)], axis=1)
        o_ref[...] = enc.T.reshape(NP_PAD, BLK_E // 128, 128)

    return pl.pallas_call(
        body,
        grid=(BATCH // BLK_E,),
        in_specs=[
            pl.BlockSpec((BLK_E, N_PMTS), lambda i: (i, 0)),
            pl.BlockSpec((BLK_E, N_PMTS), lambda i: (i, 0)),
            pl.BlockSpec((1, N_PMTS), lambda i: (0, 0)),
        ],
        out_specs=pl.BlockSpec((NP_PAD, BLK_E // 128, 128), lambda i: (0, i, 0)),
        out_shape=jax.ShapeDtypeStruct((NP_PAD, BATCH // 128, 128), jnp.int32),
    )(pred, observed, coef_row)


def _sc_stage(enc, lut):
    mesh = plsc.VectorSubcoreMesh(core_axis_name="c", subcore_axis_name="s")

    @functools.partial(
        pl.kernel,
        out_type=jax.ShapeDtypeStruct((NW, 128, 128), jnp.float32),
        mesh=mesh,
        scratch_types=[
            pltpu.VMEM((M * M // 128, 128), jnp.float32),
            pltpu.VMEM((BATCH // 128, 128), jnp.int32),
            pltpu.VMEM((BATCH // 128, 128), jnp.float32),
        ],
        compiler_params=pltpu.CompilerParams(
            needs_layout_passes=False, use_tc_tiling_on_sc=True),
    )
    def body(enc_hbm, lut_hbm, out_hbm, tab_v, enc_v, acc_v):
        wid = lax.axis_index("s") * 2 + lax.axis_index("c")

        for k in range(K_MAX):
            p = wid + k * NW

            @pl.when(p < N_PMTS)
            def _():
                pltpu.sync_copy(lut_hbm.at[p], tab_v)
                pltpu.sync_copy(enc_hbm.at[p], enc_v)

                @plsc.parallel_loop(0, BATCH // 128, unroll=2)
                def _row(rr):
                    for cc in range(8):
                        v = enc_v[rr, pl.ds(cc * L, L)]
                        is_lut = v < FB_FLAG
                        idxv = jnp.minimum(v, FB_FLAG - 1)
                        val = plsc.load_gather(tab_v, [idxv >> 7, idxv & 127])
                        fbv = plsc.bitcast(v << 16, jnp.float32)
                        r = jnp.where(is_lut, val, fbv)
                        if k == 0:
                            acc_v[rr, pl.ds(cc * L, L)] = r
                        else:
                            acc_v[rr, pl.ds(cc * L, L)] = (
                                acc_v[rr, pl.ds(cc * L, L)] + r)

        pltpu.sync_copy(acc_v, out_hbm.at[wid])

    return body(enc, lut)


def _tc_reduce(partials):
    def body(p_ref, o_ref):
        o_ref[...] = jnp.sum(p_ref[...], axis=0)

    return pl.pallas_call(
        body,
        out_shape=jax.ShapeDtypeStruct((128, 128), jnp.float32),
    )(partials)


def kernel(pred, observed, std, L_table):
    lut = L_table.reshape(N_PMTS, M * M // 128, 128)
    coef = (1.0 + P_DPE) ** 2 + P_DPE * (1.0 - P_DPE) + std * std
    enc = _tc_encode(pred, observed, coef.reshape(1, N_PMTS))
    partials = _sc_stage(enc, lut)
    return _tc_reduce(partials).reshape(BATCH)


# speedup vs baseline: 2.1876x; 1.4189x over previous
import functools

import jax
import jax.numpy as jnp
from jax import lax
from jax.experimental import pallas as pl
from jax.experimental.pallas import tpu as pltpu
from jax.experimental.pallas import tpu_sc as plsc

N_PMTS = 494
BATCH = 16384
M = 256
SWITCHING_SIGNAL = 50.0
P_DPE = 0.2
NAN_SAFE_VALUE = 1.0e6

NP_PAD = 512
NW = 32
K_MAX = (N_PMTS + NW - 1) // NW
L = 16
STEPS = BATCH // L
BLK_E = 1024
FB_FLAG = 1 << 16


def _tc_encode(pred, observed, coef_row):
    def body(p_ref, x_ref, c_ref, o_ref):
        x = x_ref[...]
        mu = jnp.maximum(p_ref[...], 1e-6)
        c = c_ref[...]

        tx = x / SWITCHING_SIGNAL * float(M)
        tx = jnp.minimum(jnp.maximum(tx, 0.0), float(M - 1))
        xi = tx.astype(jnp.int32)
        tm = mu / SWITCHING_SIGNAL * float(M)
        tm = jnp.minimum(jnp.maximum(tm, 0.0), float(M - 1))
        mi = tm.astype(jnp.int32)
        idx = xi * M + mi

        var = jnp.maximum(mu * c, 1e-12)
        d = x - mu * (1.0 + P_DPE)
        g = (d * d) / var
        nan = (x != x) | (mu != mu)
        fb = jnp.where(nan, NAN_SAFE_VALUE, g)
        fbb = lax.bitcast_convert_type(fb.astype(jnp.bfloat16), jnp.uint16)
        fbb = fbb.astype(jnp.int32) | FB_FLAG

        use_lut = (x < SWITCHING_SIGNAL) & (mu < SWITCHING_SIGNAL)
        enc = jnp.where(use_lut, idx, fbb)
        enc = jnp.concatenate(
            [enc, jnp.zeros((BLK_E, NP_PAD - N_PMTS), jnp.int32)], axis=1)
        o_ref[...] = enc.T.reshape(NP_PAD, BLK_E // 128, 128)

    return pl.pallas_call(
        body,
        grid=(BATCH // BLK_E,),
        in_specs=[
            pl.BlockSpec((BLK_E, N_PMTS), lambda i: (i, 0)),
            pl.BlockSpec((BLK_E, N_PMTS), lambda i: (i, 0)),
            pl.BlockSpec((1, N_PMTS), lambda i: (0, 0)),
        ],
        out_specs=pl.BlockSpec((NP_PAD, BLK_E // 128, 128), lambda i: (0, i, 0)),
        out_shape=jax.ShapeDtypeStruct((NP_PAD, BATCH // 128, 128), jnp.int32),
    )(pred, observed, coef_row)


def _sc_stage(enc, lut):
    mesh = plsc.VectorSubcoreMesh(core_axis_name="c", subcore_axis_name="s")

    @functools.partial(
        pl.kernel,
        out_type=jax.ShapeDtypeStruct((NW, 128, 128), jnp.float32),
        mesh=mesh,
        scratch_types=[
            pltpu.VMEM((M, M), jnp.float32),
            pltpu.VMEM((BATCH // 128, 128), jnp.int32),
            pltpu.VMEM((BATCH // 128, 128), jnp.float32),
        ],
        compiler_params=pltpu.CompilerParams(
            needs_layout_passes=False, use_tc_tiling_on_sc=True),
    )
    def body(enc_hbm, lut_hbm, out_hbm, tab_v, enc_v, acc_v):
        wid = lax.axis_index("s") * 2 + lax.axis_index("c")

        for k in range(K_MAX):
            p = wid + k * NW

            @pl.when(p < N_PMTS)
            def _():
                pltpu.sync_copy(lut_hbm.at[p], tab_v)
                pltpu.sync_copy(enc_hbm.at[p], enc_v)

                @plsc.parallel_loop(0, BATCH // 128, unroll=2)
                def _row(rr):
                    for cc in range(8):
                        v = enc_v[rr, pl.ds(cc * L, L)]
                        is_lut = v < FB_FLAG
                        idxv = jnp.minimum(v, FB_FLAG - 1)
                        val = plsc.load_gather(tab_v, [idxv >> 8, idxv & 255])
                        fbv = plsc.bitcast(v << 16, jnp.float32)
                        r = jnp.where(is_lut, val, fbv)
                        if k == 0:
                            acc_v[rr, pl.ds(cc * L, L)] = r
                        else:
                            acc_v[rr, pl.ds(cc * L, L)] = (
                                acc_v[rr, pl.ds(cc * L, L)] + r)

        pltpu.sync_copy(acc_v, out_hbm.at[wid])

    return body(enc, lut)


def _tc_reduce(partials):
    def body(p_ref, o_ref):
        o_ref[...] = jnp.sum(p_ref[...], axis=0)

    return pl.pallas_call(
        body,
        out_shape=jax.ShapeDtypeStruct((128, 128), jnp.float32),
    )(partials)


def kernel(pred, observed, std, L_table):
    lut = L_table
    coef = (1.0 + P_DPE) ** 2 + P_DPE * (1.0 - P_DPE) + std * std
    enc = _tc_encode(pred, observed, coef.reshape(1, N_PMTS))
    partials = _sc_stage(enc, lut)
    return _tc_reduce(partials).reshape(BATCH)


# speedup vs baseline: 2.1981x; 1.0048x over previous
import functools

import jax
import jax.numpy as jnp
from jax import lax
from jax.experimental import pallas as pl
from jax.experimental.pallas import tpu as pltpu
from jax.experimental.pallas import tpu_sc as plsc

N_PMTS = 494
BATCH = 16384
M = 256
SWITCHING_SIGNAL = 50.0
P_DPE = 0.2
NAN_SAFE_VALUE = 1.0e6

GRP = 256
NW = 32
K_MAX = GRP // NW
L = 16
BLK_E = 2048
FB_FLAG = 1 << 16


def _tc_encode(pred, observed, coef_row, g):
    def body(p_ref, x_ref, c_ref, o_ref):
        x = x_ref[...]
        mu = jnp.maximum(p_ref[...], 1e-6)
        c = c_ref[...]

        tx = x / SWITCHING_SIGNAL * float(M)
        tx = jnp.minimum(jnp.maximum(tx, 0.0), float(M - 1))
        xi = tx.astype(jnp.int32)
        tm = mu / SWITCHING_SIGNAL * float(M)
        tm = jnp.minimum(jnp.maximum(tm, 0.0), float(M - 1))
        mi = tm.astype(jnp.int32)
        idx = xi * M + mi

        var = jnp.maximum(mu * c, 1e-12)
        d = x - mu * (1.0 + P_DPE)
        gs = (d * d) / var
        nan = (x != x) | (mu != mu)
        fb = jnp.where(nan, NAN_SAFE_VALUE, gs)
        fbb = lax.bitcast_convert_type(fb.astype(jnp.bfloat16), jnp.uint16)
        fbb = fbb.astype(jnp.int32) | FB_FLAG

        use_lut = (x < SWITCHING_SIGNAL) & (mu < SWITCHING_SIGNAL)
        enc = jnp.where(use_lut, idx, fbb)
        o_ref[...] = enc.T.reshape(GRP, BLK_E // 128, 128)

    return pl.pallas_call(
        body,
        grid=(BATCH // BLK_E,),
        in_specs=[
            pl.BlockSpec((BLK_E, GRP), lambda i: (i, g)),
            pl.BlockSpec((BLK_E, GRP), lambda i: (i, g)),
            pl.BlockSpec((1, GRP), lambda i: (0, g)),
        ],
        out_specs=pl.BlockSpec((GRP, BLK_E // 128, 128), lambda i: (0, i, 0)),
        out_shape=jax.ShapeDtypeStruct((GRP, BATCH // 128, 128), jnp.int32),
    )(pred, observed, coef_row)


def _sc_stage(enc, lut, g, n_valid):
    mesh = plsc.VectorSubcoreMesh(core_axis_name="c", subcore_axis_name="s")

    @functools.partial(
        pl.kernel,
        out_type=jax.ShapeDtypeStruct((NW, 128, 128), jnp.float32),
        mesh=mesh,
        scratch_types=[
            pltpu.VMEM((M, M), jnp.float32),
            pltpu.VMEM((BATCH // 128, 128), jnp.int32),
            pltpu.VMEM((BATCH // 128, 128), jnp.float32),
        ],
        compiler_params=pltpu.CompilerParams(
            needs_layout_passes=False, use_tc_tiling_on_sc=True),
    )
    def body(enc_hbm, lut_hbm, out_hbm, tab_v, enc_v, acc_v):
        wid = lax.axis_index("s") * 2 + lax.axis_index("c")

        for k in range(K_MAX):
            p = wid + k * NW

            @pl.when(p < n_valid)
            def _():
                pltpu.sync_copy(lut_hbm.at[g * GRP + p], tab_v)
                pltpu.sync_copy(enc_hbm.at[p], enc_v)

                @plsc.parallel_loop(0, BATCH // 128, unroll=2)
                def _row(rr):
                    for cc in range(8):
                        v = enc_v[rr, pl.ds(cc * L, L)]
                        is_lut = v < FB_FLAG
                        idxv = jnp.minimum(v, FB_FLAG - 1)
                        val = plsc.load_gather(tab_v, [idxv >> 8, idxv & 255])
                        fbv = plsc.bitcast(v << 16, jnp.float32)
                        r = jnp.where(is_lut, val, fbv)
                        if k == 0:
                            acc_v[rr, pl.ds(cc * L, L)] = r
                        else:
                            acc_v[rr, pl.ds(cc * L, L)] = (
                                acc_v[rr, pl.ds(cc * L, L)] + r)

        pltpu.sync_copy(acc_v, out_hbm.at[wid])

    return body(enc, lut)


def _tc_reduce(pa, pb):
    def body(a_ref, b_ref, o_ref):
        o_ref[...] = jnp.sum(a_ref[...], axis=0) + jnp.sum(b_ref[...], axis=0)

    return pl.pallas_call(
        body,
        out_shape=jax.ShapeDtypeStruct((128, 128), jnp.float32),
    )(pa, pb)


def kernel(pred, observed, std, L_table):
    coef = (1.0 + P_DPE) ** 2 + P_DPE * (1.0 - P_DPE) + std * std
    coef_row = coef.reshape(1, N_PMTS)
    enc_a = _tc_encode(pred, observed, coef_row, 0)
    part_a = _sc_stage(enc_a, L_table, 0, GRP)
    enc_b = _tc_encode(pred, observed, coef_row, 1)
    part_b = _sc_stage(enc_b, L_table, 1, N_PMTS - GRP)
    return _tc_reduce(part_a, part_b).reshape(BATCH)


# speedup vs baseline: 2.2593x; 1.0278x over previous
import functools

import jax
import jax.numpy as jnp
from jax import lax
from jax.experimental import pallas as pl
from jax.experimental.pallas import tpu as pltpu
from jax.experimental.pallas import tpu_sc as plsc

N_PMTS = 494
BATCH = 16384
M = 256
SWITCHING_SIGNAL = 50.0
P_DPE = 0.2
NAN_SAFE_VALUE = 1.0e6

GRP = 256
NW = 32
K_MAX = GRP // NW
L = 16
BLK_E = 2048
FB_FLAG = 1 << 16


def _tc_encode(pred, observed, coef_row, g):
    def body(p_ref, x_ref, c_ref, o_ref):
        x = x_ref[...]
        mu = jnp.maximum(p_ref[...], 1e-6)
        c = c_ref[...]

        tx = x / SWITCHING_SIGNAL * float(M)
        tx = jnp.minimum(jnp.maximum(tx, 0.0), float(M - 1))
        xi = tx.astype(jnp.int32)
        tm = mu / SWITCHING_SIGNAL * float(M)
        tm = jnp.minimum(jnp.maximum(tm, 0.0), float(M - 1))
        mi = tm.astype(jnp.int32)
        idx = xi * M + mi

        var = jnp.maximum(mu * c, 1e-12)
        d = x - mu * (1.0 + P_DPE)
        gs = (d * d) / var
        nan = (x != x) | (mu != mu)
        fb = jnp.where(nan, NAN_SAFE_VALUE, gs)
        fbb = lax.bitcast_convert_type(fb.astype(jnp.bfloat16), jnp.uint16)
        fbb = fbb.astype(jnp.int32) | FB_FLAG

        use_lut = (x < SWITCHING_SIGNAL) & (mu < SWITCHING_SIGNAL)
        enc = jnp.where(use_lut, idx, fbb)
        o_ref[...] = enc.T.reshape(GRP, BLK_E // 128, 128)

    return pl.pallas_call(
        body,
        grid=(BATCH // BLK_E,),
        in_specs=[
            pl.BlockSpec((BLK_E, GRP), lambda i: (i, g)),
            pl.BlockSpec((BLK_E, GRP), lambda i: (i, g)),
            pl.BlockSpec((1, GRP), lambda i: (0, g)),
        ],
        out_specs=pl.BlockSpec((GRP, BLK_E // 128, 128), lambda i: (0, i, 0)),
        out_shape=jax.ShapeDtypeStruct((GRP, BATCH // 128, 128), jnp.int32),
    )(pred, observed, coef_row)


def _sc_stage(enc, lut, g, n_valid):
    mesh = plsc.VectorSubcoreMesh(core_axis_name="c", subcore_axis_name="s")

    @functools.partial(
        pl.kernel,
        out_type=jax.ShapeDtypeStruct((NW, 128, 128), jnp.float32),
        mesh=mesh,
        scratch_types=[
            pltpu.VMEM((M, M), jnp.float32),
            pltpu.VMEM((BATCH // 128, 128), jnp.int32),
            pltpu.VMEM((BATCH // 128, 128), jnp.int32),
            pltpu.VMEM((BATCH // 128, 128), jnp.float32),
            pltpu.SemaphoreType.DMA,
        ],
        compiler_params=pltpu.CompilerParams(
            needs_layout_passes=False, use_tc_tiling_on_sc=True),
    )
    def body(enc_hbm, lut_hbm, out_hbm, tab_v, enc_v0, enc_v1, acc_v, sem):
        wid = lax.axis_index("s") * 2 + lax.axis_index("c")
        bufs = (enc_v0, enc_v1)

        @pl.when(wid < n_valid)
        def _prefetch():
            pltpu.async_copy(enc_hbm.at[wid], enc_v0, sem)

        for k in range(K_MAX):
            p = wid + k * NW
            enc_v = bufs[k % 2]
            nxt_v = bufs[(k + 1) % 2]

            @pl.when(p < n_valid)
            def _():
                pltpu.make_async_copy(enc_hbm.at[p], enc_v, sem).wait()
                if k + 1 < K_MAX:
                    @pl.when(p + NW < n_valid)
                    def _():
                        pltpu.async_copy(enc_hbm.at[p + NW], nxt_v, sem)
                pltpu.sync_copy(lut_hbm.at[g * GRP + p], tab_v)

                @plsc.parallel_loop(0, BATCH // 128, unroll=2)
                def _row(rr):
                    for cc in range(8):
                        v = enc_v[rr, pl.ds(cc * L, L)]
                        is_lut = v < FB_FLAG
                        idxv = jnp.minimum(v, FB_FLAG - 1)
                        val = plsc.load_gather(tab_v, [idxv >> 8, idxv & 255])
                        fbv = plsc.bitcast(v << 16, jnp.float32)
                        r = jnp.where(is_lut, val, fbv)
                        if k == 0:
                            acc_v[rr, pl.ds(cc * L, L)] = r
                        else:
                            acc_v[rr, pl.ds(cc * L, L)] = (
                                acc_v[rr, pl.ds(cc * L, L)] + r)

        pltpu.sync_copy(acc_v, out_hbm.at[wid])

    return body(enc, lut)


def _tc_reduce(pa, pb):
    def body(a_ref, b_ref, o_ref):
        o_ref[...] = jnp.sum(a_ref[...], axis=0) + jnp.sum(b_ref[...], axis=0)

    return pl.pallas_call(
        body,
        out_shape=jax.ShapeDtypeStruct((128, 128), jnp.float32),
    )(pa, pb)


def kernel(pred, observed, std, L_table):
    coef = (1.0 + P_DPE) ** 2 + P_DPE * (1.0 - P_DPE) + std * std
    coef_row = coef.reshape(1, N_PMTS)
    enc_a = _tc_encode(pred, observed, coef_row, 0)
    part_a = _sc_stage(enc_a, L_table, 0, GRP)
    enc_b = _tc_encode(pred, observed, coef_row, 1)
    part_b = _sc_stage(enc_b, L_table, 1, N_PMTS - GRP)
    return _tc_reduce(part_a, part_b).reshape(BATCH)
